# baseline (speedup 1.0000x reference)
import functools
import math

import jax
import jax.numpy as jnp
from jax import lax
from jax.experimental import pallas as pl
from jax.experimental.pallas import tpu as pltpu
from jax.experimental.pallas import tpu_sc as plsc

_SMOOTHING = 0.1
_CONFIDENCE = 1.0 - _SMOOTHING
_N = 1024
_V = 100000
_EPS = _SMOOTHING / (_V - 1)
_CONST = _N * ((_V - 1) * _EPS * math.log(_EPS) + _CONFIDENCE * math.log(_CONFIDENCE))

_NW = 32
_L = 16
_PER = _N // _NW

_CS = 65536
_CW = 4096
_CH_PER_GRP = _CS // _CW
_GRP = _PER // 8
_NCH = _GRP * _CH_PER_GRP

_RB = 64
_NB = _N // _RB
_WB = 8192
_C0 = _CS // _WB
_NCB = -(-(_V - _CS) // _WB)


def _chunk_src(x_hbm, wid, kk):
    grp = kk // _CH_PER_GRP
    col = (kk % _CH_PER_GRP) * _CW
    row8 = (wid * _GRP + grp) * 8
    return x_hbm.at[pl.ds(row8, 8), pl.ds(col, _CW)]


def _reduce_chunk(buf, a0, a1):
    for r in range(8):
        @plsc.parallel_loop(0, _CW // 32, unroll=10, carry=(a0, a1))
        def body(i, ab, r=r):
            b0, b1 = ab
            o = i * 32
            b0 = b0 + buf[r, pl.ds(o, _L)]
            b1 = b1 + buf[r, pl.ds(o + _L, _L)]
            return (b0, b1)

        a0, a1 = body
    return a0, a1


def _sc_body(x_hbm, tgt_hbm, out_hbm, buf0, buf1, gbuf0, gbuf1, tv,
             accv, sem0, sem1, gsem0, gsem1):
    wid = lax.axis_index("s") * 2 + lax.axis_index("c")
    base = wid * _PER

    pltpu.sync_copy(tgt_hbm.at[pl.ds(base, _PER)], tv)

    tva = tv[pl.ds(0, _L)]
    tvb = tv[pl.ds(_L, _L)]

    ghandles = []
    for k in range(_PER):
        t = (tva if k < _L else tvb)[k % _L]
        c0 = pl.multiple_of(t & (-128), 128)
        row8 = base + 8 * (k // 8)
        gbuf = gbuf0 if k < _L else gbuf1
        gsem = gsem0 if k < _L else gsem1
        ghandles.append(
            pltpu.async_copy(
                x_hbm.at[pl.ds(row8, 8), pl.ds(c0, 128)],
                gbuf.at[pl.ds(8 * (k % _L), 8), :],
                gsem,
            )
        )

    pltpu.async_copy(_chunk_src(x_hbm, wid, 0), buf0, sem0)
    pltpu.async_copy(_chunk_src(x_hbm, wid, 1), buf1, sem1)
    accv[...] = jnp.zeros((3 * _L,), jnp.float32)

    def outer(k2, _):
        bufs = (buf0, buf1)
        sems = (sem0, sem1)
        for b in range(2):
            kk = 2 * k2 + b
            pltpu.make_async_copy(_chunk_src(x_hbm, wid, kk), bufs[b], sems[b]).wait()
            nxt = kk + 2

            @pl.when(nxt < _NCH)
            def _issue(b=b, nxt=nxt):
                pltpu.async_copy(_chunk_src(x_hbm, wid, nxt), bufs[b], sems[b])

            a0 = accv[pl.ds(0, _L)]
            a1 = accv[pl.ds(_L, _L)]
            a0, a1 = _reduce_chunk(bufs[b], a0, a1)
            accv[pl.ds(0, _L)] = a0
            accv[pl.ds(_L, _L)] = a1
        return 0

    lax.fori_loop(0, _NCH // 2, outer, 0)

    for h in ghandles:
        h.wait()
    iota = lax.iota(jnp.int32, _L)
    gacc = jnp.zeros((_L,), jnp.float32)
    for k in range(_PER):
        t = (tva if k < _L else tvb)[k % _L]
        tmod = t & 127
        gbuf = gbuf0 if k < _L else gbuf1
        row = 8 * (k % _L) + (k % 8)
        for j in range(8):
            v = gbuf[row, pl.ds(_L * j, _L)]
            gacc = gacc + jnp.where(iota + _L * j == tmod, v, 0.0)
    accv[pl.ds(2 * _L, _L)] = gacc

    pltpu.sync_copy(accv, out_hbm.at[wid])


_sc_call = functools.partial(
    pl.kernel,
    mesh=plsc.VectorSubcoreMesh(core_axis_name="c", subcore_axis_name="s"),
    out_type=jax.ShapeDtypeStruct((_NW, 3 * _L), jnp.float32),
    scratch_types=[
        pltpu.VMEM((8, _CW), jnp.float32),
        pltpu.VMEM((8, _CW), jnp.float32),
        pltpu.VMEM((8 * _L, 128), jnp.float32),
        pltpu.VMEM((8 * _L, 128), jnp.float32),
        pltpu.VMEM((_PER,), jnp.int32),
        pltpu.VMEM((3 * _L,), jnp.float32),
        pltpu.SemaphoreType.DMA,
        pltpu.SemaphoreType.DMA,
        pltpu.SemaphoreType.DMA,
        pltpu.SemaphoreType.DMA,
    ],
)(_sc_body)


def _tc_body(x_ref, out_ref):
    b = pl.program_id(0)
    c = pl.program_id(1)

    @pl.when((b == 0) & (c == 0))
    def _init():
        out_ref[...] = jnp.zeros((1, 1), jnp.float32)

    xb = x_ref[...]

    @pl.when(c < _NCB - 1)
    def _full():
        out_ref[...] += jnp.sum(xb).reshape(1, 1)

    @pl.when(c == _NCB - 1)
    def _masked():
        col = (_C0 + c) * _WB + lax.broadcasted_iota(jnp.int32, (_RB, _WB), 1)
        out_ref[...] += jnp.sum(jnp.where(col < _V, xb, 0.0)).reshape(1, 1)


def kernel(x, target):
    tgt = target.astype(jnp.int32)
    parts = _sc_call(x, tgt)
    tc = pl.pallas_call(
        _tc_body,
        grid=(_NB, _NCB),
        in_specs=[pl.BlockSpec((_RB, _WB), lambda b, c: (b, c + _C0))],
        out_specs=pl.BlockSpec((1, 1), lambda b, c: (0, 0)),
        out_shape=jax.ShapeDtypeStruct((1, 1), jnp.float32),
        compiler_params=pltpu.CompilerParams(
            dimension_semantics=("arbitrary", "arbitrary"),
        ),
    )(x)
    dense = jnp.sum(parts[:, : 2 * _L]) + tc[0, 0]
    gath = jnp.sum(parts[:, 2 * _L :])
    return (
        jnp.float32(_CONST)
        - jnp.float32(_EPS) * dense
        - jnp.float32(_CONFIDENCE - _EPS) * gath
    )


# speedup vs baseline: 1.0738x; 1.0738x over previous
import functools
import math

import jax
import jax.numpy as jnp
from jax import lax
from jax.experimental import pallas as pl
from jax.experimental.pallas import tpu as pltpu
from jax.experimental.pallas import tpu_sc as plsc

_SMOOTHING = 0.1
_CONFIDENCE = 1.0 - _SMOOTHING
_N = 1024
_V = 100000
_EPS = _SMOOTHING / (_V - 1)
_CONST = _N * ((_V - 1) * _EPS * math.log(_EPS) + _CONFIDENCE * math.log(_CONFIDENCE))

_NW = 32
_L = 16
_PER = _N // _NW

_CW = 4096
_CH_PER_GRP = 24
_GRP = _PER // 8
_NCH = _GRP * _CH_PER_GRP
_REM0 = _CH_PER_GRP * _CW
_RW = 1792
_RV = _V - _REM0


def _chunk_src(x_hbm, wid, kk):
    grp = kk // _CH_PER_GRP
    col = (kk % _CH_PER_GRP) * _CW
    row8 = (wid * _GRP + grp) * 8
    return x_hbm.at[pl.ds(row8, 8), pl.ds(col, _CW)]


def _rem_src(x_hbm, wid, g):
    row8 = (wid * _GRP + g) * 8
    col = pl.multiple_of(_REM0 + 0 * row8, 128)
    return x_hbm.at[pl.ds(row8, 8), pl.ds(col, _RW)]


def _reduce_chunk(buf, acc, cols, unroll=8):
    for r in range(8):
        @plsc.parallel_loop(0, cols // 64, unroll=unroll, carry=acc)
        def body(i, ab, r=r):
            b0, b1, b2, b3 = ab
            o = i * 64
            b0 = b0 + buf[r, pl.ds(o, _L)]
            b1 = b1 + buf[r, pl.ds(o + _L, _L)]
            b2 = b2 + buf[r, pl.ds(o + 2 * _L, _L)]
            b3 = b3 + buf[r, pl.ds(o + 3 * _L, _L)]
            return (b0, b1, b2, b3)

        acc = body
    return acc


def _load_acc(accv):
    return tuple(accv[pl.ds(j * _L, _L)] for j in range(4))


def _store_acc(accv, acc):
    for j in range(4):
        accv[pl.ds(j * _L, _L)] = acc[j]


def _sc_body(x_hbm, tgt_hbm, out_hbm, buf0, buf1, gbuf, tv, accv,
             sem0, sem1, gsem):
    wid = lax.axis_index("s") * 2 + lax.axis_index("c")
    base = wid * _PER

    pltpu.sync_copy(tgt_hbm.at[pl.ds(base, _PER)], tv)
    tva = tv[pl.ds(0, _L)]
    tvb = tv[pl.ds(_L, _L)]

    def issue_gather(k):
        t = pl.multiple_of((tva if k < _L else tvb)[k % _L] & (-128), 128)
        row8 = base + 8 * (k // 8)
        return pltpu.async_copy(
            x_hbm.at[pl.ds(row8, 8), pl.ds(t, 128)],
            gbuf.at[pl.ds(8 * (k % _L), 8), :],
            gsem,
        )

    def extract_gather(k, gacc):
        t = (tva if k < _L else tvb)[k % _L]
        tmod = t & 127
        row = 8 * (k % _L) + (k % 8)
        iota = lax.iota(jnp.int32, _L)
        for j in range(8):
            v = gbuf[row, pl.ds(_L * j, _L)]
            gacc = gacc + jnp.where(iota + _L * j == tmod, v, 0.0)
        return gacc

    gh0 = [issue_gather(k) for k in range(_L)]

    pltpu.async_copy(_chunk_src(x_hbm, wid, 0), buf0, sem0)
    pltpu.async_copy(_chunk_src(x_hbm, wid, 1), buf1, sem1)
    accv[...] = jnp.zeros((5 * _L,), jnp.float32)

    def outer(k2, _):
        bufs = (buf0, buf1)
        sems = (sem0, sem1)
        for b in range(2):
            kk = 2 * k2 + b
            pltpu.make_async_copy(_chunk_src(x_hbm, wid, kk), bufs[b], sems[b]).wait()
            nxt = kk + 2

            @pl.when(nxt < _NCH)
            def _issue(b=b, nxt=nxt):
                pltpu.async_copy(_chunk_src(x_hbm, wid, nxt), bufs[b], sems[b])

            acc = _reduce_chunk(bufs[b], _load_acc(accv), _CW)
            _store_acc(accv, acc)
        return 0

    lax.fori_loop(0, _NCH // 2, outer, 0)

    for h in gh0:
        h.wait()
    gacc = jnp.zeros((_L,), jnp.float32)
    for k in range(_L):
        gacc = extract_gather(k, gacc)
    gh1 = [issue_gather(k) for k in range(_L, _PER)]

    pltpu.async_copy(_rem_src(x_hbm, wid, 0), buf0.at[:, pl.ds(0, _RW)], sem0)
    pltpu.async_copy(_rem_src(x_hbm, wid, 1), buf1.at[:, pl.ds(0, _RW)], sem1)
    for g in range(_GRP):
        buf = buf0 if g % 2 == 0 else buf1
        sem = sem0 if g % 2 == 0 else sem1
        pltpu.make_async_copy(_rem_src(x_hbm, wid, g), buf.at[:, pl.ds(0, _RW)], sem).wait()
        acc = _reduce_chunk(buf, _load_acc(accv), _RV - 32, unroll=13)
        a0, a1, a2, a3 = acc
        for r in range(8):
            a0 = a0 + buf[r, pl.ds(_RV - 32, _L)]
            a1 = a1 + buf[r, pl.ds(_RV - _L, _L)]
        _store_acc(accv, (a0, a1, a2, a3))
        if g + 2 < _GRP:
            pltpu.async_copy(_rem_src(x_hbm, wid, g + 2), buf.at[:, pl.ds(0, _RW)], sem)

    for h in gh1:
        h.wait()
    for k in range(_L, _PER):
        gacc = extract_gather(k, gacc)
    accv[pl.ds(4 * _L, _L)] = gacc

    pltpu.sync_copy(accv, out_hbm.at[wid])


_sc_call = functools.partial(
    pl.kernel,
    mesh=plsc.VectorSubcoreMesh(core_axis_name="c", subcore_axis_name="s"),
    out_type=jax.ShapeDtypeStruct((_NW, 5 * _L), jnp.float32),
    scratch_types=[
        pltpu.VMEM((8, _CW), jnp.float32),
        pltpu.VMEM((8, _CW), jnp.float32),
        pltpu.VMEM((8 * _L, 128), jnp.float32),
        pltpu.VMEM((_PER,), jnp.int32),
        pltpu.VMEM((5 * _L,), jnp.float32),
        pltpu.SemaphoreType.DMA,
        pltpu.SemaphoreType.DMA,
        pltpu.SemaphoreType.DMA,
    ],
)(_sc_body)


def _combine_body(p_ref, out_ref):
    p = p_ref[...]
    col = lax.broadcasted_iota(jnp.int32, (_NW, 5 * _L), 1)
    w = jnp.where(col < 4 * _L, jnp.float32(_EPS), jnp.float32(_CONFIDENCE - _EPS))
    out_ref[...] = (jnp.float32(_CONST) - jnp.sum(p * w)).reshape(1, 1)


def kernel(x, target):
    tgt = target.astype(jnp.int32)
    parts = _sc_call(x, tgt)
    out = pl.pallas_call(
        _combine_body,
        in_specs=[pl.BlockSpec((_NW, 5 * _L), lambda: (0, 0))],
        out_specs=pl.BlockSpec((1, 1), lambda: (0, 0)),
        out_shape=jax.ShapeDtypeStruct((1, 1), jnp.float32),
    )(parts)
    return out[0, 0]


# speedup vs baseline: 1.2050x; 1.1221x over previous
import functools
import math

import jax
import jax.numpy as jnp
from jax import lax
from jax.experimental import pallas as pl
from jax.experimental.pallas import tpu as pltpu
from jax.experimental.pallas import tpu_sc as plsc

_SMOOTHING = 0.1
_CONFIDENCE = 1.0 - _SMOOTHING
_N = 1024
_V = 100000
_EPS = _SMOOTHING / (_V - 1)
_CONST = _N * ((_V - 1) * _EPS * math.log(_EPS) + _CONFIDENCE * math.log(_CONFIDENCE))

_NW = 32
_L = 16
_PER = _N // _NW

_RB = 32
_NB = _N // _RB


def _sc_body(x_hbm, tgt_hbm, out_hbm, gbuf, tv, accv, gsem):
    wid = lax.axis_index("s") * 2 + lax.axis_index("c")
    base = wid * _PER

    pltpu.sync_copy(tgt_hbm.at[pl.ds(base, _PER)], tv)
    tva = tv[pl.ds(0, _L)]
    tvb = tv[pl.ds(_L, _L)]

    def scalar_t(k):
        return (tva if k < _L else tvb)[k % _L]

    handles = []
    for k in range(_PER):
        t = pl.multiple_of(scalar_t(k) & (-128), 128)
        row8 = base + 8 * (k // 8)
        handles.append(
            pltpu.async_copy(
                x_hbm.at[pl.ds(row8, 8), pl.ds(t, 128)],
                gbuf.at[pl.ds(8 * k, 8), :],
                gsem,
            )
        )
    for h in handles:
        h.wait()

    iota = lax.iota(jnp.int32, _L)
    gacc = jnp.zeros((_L,), jnp.float32)
    for k in range(_PER):
        tmod = scalar_t(k) & 127
        row = 8 * k + (k % 8)
        for j in range(8):
            v = gbuf[row, pl.ds(_L * j, _L)]
            gacc = gacc + jnp.where(iota + _L * j == tmod, v, 0.0)
    accv[...] = gacc
    pltpu.sync_copy(accv, out_hbm.at[wid])


_sc_call = functools.partial(
    pl.kernel,
    mesh=plsc.VectorSubcoreMesh(core_axis_name="c", subcore_axis_name="s"),
    out_type=jax.ShapeDtypeStruct((_NW, _L), jnp.float32),
    scratch_types=[
        pltpu.VMEM((8 * _PER, 128), jnp.float32),
        pltpu.VMEM((_PER,), jnp.int32),
        pltpu.VMEM((_L,), jnp.float32),
        pltpu.SemaphoreType.DMA,
    ],
)(_sc_body)


def _tc_body(x_ref, out_ref):
    b = pl.program_id(0)

    @pl.when(b == 0)
    def _init():
        out_ref[...] = jnp.zeros((1, 1), jnp.float32)

    out_ref[...] += jnp.sum(x_ref[...]).reshape(1, 1)


def _combine_body(g_ref, s_ref, out_ref):
    gsum = jnp.sum(g_ref[...])
    out_ref[...] = (
        jnp.float32(_CONST)
        - jnp.float32(_EPS) * s_ref[0, 0]
        - jnp.float32(_CONFIDENCE - _EPS) * gsum
    ).reshape(1, 1)


def kernel(x, target):
    tgt = target.astype(jnp.int32)
    parts = _sc_call(x, tgt)
    s = pl.pallas_call(
        _tc_body,
        grid=(_NB,),
        in_specs=[pl.BlockSpec((_RB, _V), lambda b: (b, 0))],
        out_specs=pl.BlockSpec((1, 1), lambda b: (0, 0)),
        out_shape=jax.ShapeDtypeStruct((1, 1), jnp.float32),
        compiler_params=pltpu.CompilerParams(
            dimension_semantics=("arbitrary",),
        ),
    )(x)
    out = pl.pallas_call(
        _combine_body,
        in_specs=[
            pl.BlockSpec((_NW, _L), lambda: (0, 0)),
            pl.BlockSpec((1, 1), lambda: (0, 0)),
        ],
        out_specs=pl.BlockSpec((1, 1), lambda: (0, 0)),
        out_shape=jax.ShapeDtypeStruct((1, 1), jnp.float32),
    )(parts, s)
    return out[0, 0]
